# Initial kernel scaffold; baseline (speedup 1.0000x reference)
import jax
import jax.numpy as jnp
from jax.experimental import pallas as pl


def kernel(x, y, seq_len):
    raise NotImplementedError("write your pallas kernel here")


# speedup vs baseline: 1.9835x; 1.9835x over previous
import jax
import jax.numpy as jnp
from jax.experimental import pallas as pl
from jax.experimental.pallas import tpu as pltpu

B, L, D = 16, 4096, 256
TL = 512
NB = L // TL
N = B * L


def _in_index_map(b, j, seq_ref):
    last_valid = jnp.maximum((seq_ref[b] + TL - 1) // TL - 1, 0)
    return (b, jnp.minimum(j, last_valid), 0)


def _stats_kernel(seq_ref, x_ref, y_ref, stats_ref, acc_ref):
    b = pl.program_id(0)
    j = pl.program_id(1)

    @pl.when((b == 0) & (j == 0))
    def _():
        acc_ref[...] = jnp.zeros_like(acc_ref)

    start = j * TL
    slen = seq_ref[b]
    col0 = jax.lax.broadcasted_iota(jnp.int32, (TL, D), 1) == 0

    def accumulate(xm, ym):
        acc_ref[0] += jnp.sum(xm, axis=0, keepdims=True)
        acc_ref[1] += jnp.sum(xm * xm, axis=0, keepdims=True)
        acc_ref[2] += jnp.sum(ym, axis=0, keepdims=True)
        acc_ref[3] += jnp.sum(ym * ym, axis=0, keepdims=True)

    @pl.when(start + TL <= slen)
    def _():
        xb = x_ref[0]
        yb = y_ref[0]
        ym = jnp.where(col0, xb - yb, yb)
        accumulate(xb, ym)

    @pl.when((start < slen) & (start + TL > slen))
    def _():
        xb = x_ref[0]
        yb = y_ref[0]
        rows = jax.lax.broadcasted_iota(jnp.int32, (TL, 1), 0) + start
        valid = rows < slen
        xm = jnp.where(valid, xb, 0.0)
        ym = jnp.where(valid, yb, 0.0)
        ym = jnp.where(col0, xm - ym, ym)
        accumulate(xm, ym)

    @pl.when((b == B - 1) & (j == NB - 1))
    def _():
        inv_n = 1.0 / N
        inv_nm1 = 1.0 / (N - 1)
        x_loc = acc_ref[0] * inv_n
        y_loc = acc_ref[2] * inv_n
        x_var = (acc_ref[1] - N * x_loc * x_loc) * inv_nm1
        y_var = (acc_ref[3] - N * y_loc * y_loc) * inv_nm1
        x_rs = jax.lax.rsqrt(x_var)
        y_rs = jax.lax.rsqrt(y_var)
        stats_ref[...] = jnp.concatenate(
            [x_rs, -x_loc * x_rs, y_rs, -y_loc * y_rs,
             jnp.zeros((4, D), jnp.float32)], axis=0)


def _norm_kernel(seq_ref, stats_ref, x_ref, y_ref, xo_ref, yo_ref, m_ref):
    b = pl.program_id(0)
    j = pl.program_id(1)
    start = j * TL
    slen = seq_ref[b]

    x_rs = stats_ref[0:1]
    x_fill = stats_ref[1:2]
    y_rs = stats_ref[2:3]
    y_fill = stats_ref[3:4]
    col0 = jax.lax.broadcasted_iota(jnp.int32, (TL, D), 1) == 0

    mrows = jax.lax.broadcasted_iota(jnp.int32, (1, 1, TL), 2) + start
    m_ref[...] = (mrows < slen).astype(jnp.float32)

    @pl.when(start + TL <= slen)
    def _():
        xb = x_ref[0]
        yb = y_ref[0]
        xo_ref[0] = xb * x_rs + x_fill
        ym = jnp.where(col0, xb - yb, yb)
        yo_ref[0] = ym * y_rs + y_fill

    @pl.when((start < slen) & (start + TL > slen))
    def _():
        xb = x_ref[0]
        yb = y_ref[0]
        rows = jax.lax.broadcasted_iota(jnp.int32, (TL, 1), 0) + start
        valid = rows < slen
        xo_ref[0] = jnp.where(valid, xb * x_rs + x_fill,
                              jnp.broadcast_to(x_fill, (TL, D)))
        ym = jnp.where(col0, xb - yb, yb)
        yo_ref[0] = jnp.where(valid, ym * y_rs + y_fill,
                              jnp.broadcast_to(y_fill, (TL, D)))

    @pl.when(start >= slen)
    def _():
        xo_ref[0] = jnp.broadcast_to(x_fill, (TL, D))
        yo_ref[0] = jnp.broadcast_to(y_fill, (TL, D))


def kernel(x, y, seq_len):
    seq32 = seq_len.astype(jnp.int32)

    stats = pl.pallas_call(
        _stats_kernel,
        grid_spec=pltpu.PrefetchScalarGridSpec(
            num_scalar_prefetch=1,
            grid=(B, NB),
            in_specs=[
                pl.BlockSpec((1, TL, D), _in_index_map),
                pl.BlockSpec((1, TL, D), _in_index_map),
            ],
            out_specs=pl.BlockSpec((8, D), lambda b, j, seq_ref: (0, 0)),
            scratch_shapes=[pltpu.VMEM((4, 1, D), jnp.float32)],
        ),
        out_shape=jax.ShapeDtypeStruct((8, D), jnp.float32),
        compiler_params=pltpu.CompilerParams(
            dimension_semantics=("arbitrary", "arbitrary")),
    )(seq32, x, y)

    x_out, y_out, mask_f = pl.pallas_call(
        _norm_kernel,
        grid_spec=pltpu.PrefetchScalarGridSpec(
            num_scalar_prefetch=1,
            grid=(B, NB),
            in_specs=[
                pl.BlockSpec((8, D), lambda b, j, seq_ref: (0, 0)),
                pl.BlockSpec((1, TL, D), _in_index_map),
                pl.BlockSpec((1, TL, D), _in_index_map),
            ],
            out_specs=[
                pl.BlockSpec((1, TL, D), lambda b, j, seq_ref: (b, j, 0)),
                pl.BlockSpec((1, TL, D), lambda b, j, seq_ref: (b, j, 0)),
                pl.BlockSpec((1, 1, TL),
                             lambda b, j, seq_ref: (b * NB + j, 0, 0)),
            ],
        ),
        out_shape=[
            jax.ShapeDtypeStruct((B, L, D), jnp.float32),
            jax.ShapeDtypeStruct((B, L, D), jnp.float32),
            jax.ShapeDtypeStruct((B * NB, 1, TL), jnp.float32),
        ],
        compiler_params=pltpu.CompilerParams(
            dimension_semantics=("arbitrary", "arbitrary")),
    )(seq32, stats, x, y)

    mask = mask_f.reshape(B, L).astype(bool)
    return (x_out, y_out, seq_len, mask)


# speedup vs baseline: 1.9847x; 1.0006x over previous
import jax
import jax.numpy as jnp
from jax.experimental import pallas as pl
from jax.experimental.pallas import tpu as pltpu

B, L, D = 16, 4096, 256
TL = 512
NB = L // TL
N = B * L


def _in_index_map(b, j, seq_ref):
    last_valid = jnp.maximum((seq_ref[b] + TL - 1) // TL - 1, 0)
    return (b, jnp.minimum(j, last_valid), 0)


def _stats_kernel(seq_ref, x_ref, y_ref, stats_ref, acc_ref):
    b = pl.program_id(0)
    j = pl.program_id(1)

    @pl.when((b == 0) & (j == 0))
    def _():
        acc_ref[...] = jnp.zeros_like(acc_ref)

    start = j * TL
    slen = seq_ref[b]
    col0 = jax.lax.broadcasted_iota(jnp.int32, (TL, D), 1) == 0

    def accumulate(xm, ym):
        acc_ref[0] += jnp.sum(xm, axis=0, keepdims=True)
        acc_ref[1] += jnp.sum(xm * xm, axis=0, keepdims=True)
        acc_ref[2] += jnp.sum(ym, axis=0, keepdims=True)
        acc_ref[3] += jnp.sum(ym * ym, axis=0, keepdims=True)

    @pl.when(start + TL <= slen)
    def _():
        xb = x_ref[0]
        yb = y_ref[0]
        ym = jnp.where(col0, xb - yb, yb)
        accumulate(xb, ym)

    @pl.when((start < slen) & (start + TL > slen))
    def _():
        xb = x_ref[0]
        yb = y_ref[0]
        rows = jax.lax.broadcasted_iota(jnp.int32, (TL, 1), 0) + start
        valid = rows < slen
        xm = jnp.where(valid, xb, 0.0)
        ym = jnp.where(valid, yb, 0.0)
        ym = jnp.where(col0, xm - ym, ym)
        accumulate(xm, ym)

    @pl.when((b == B - 1) & (j == NB - 1))
    def _():
        inv_n = 1.0 / N
        inv_nm1 = 1.0 / (N - 1)
        x_loc = acc_ref[0] * inv_n
        y_loc = acc_ref[2] * inv_n
        x_var = (acc_ref[1] - N * x_loc * x_loc) * inv_nm1
        y_var = (acc_ref[3] - N * y_loc * y_loc) * inv_nm1
        x_rs = jax.lax.rsqrt(x_var)
        y_rs = jax.lax.rsqrt(y_var)
        stats_ref[...] = jnp.concatenate(
            [x_rs, -x_loc * x_rs, y_rs, -y_loc * y_rs,
             jnp.zeros((4, D), jnp.float32)], axis=0)


def _norm_kernel(seq_ref, stats_ref, x_ref, y_ref, xo_ref, yo_ref, m_ref):
    b = pl.program_id(0)
    j = pl.program_id(1)
    start = j * TL
    slen = seq_ref[b]

    x_rs = stats_ref[0:1]
    x_fill = stats_ref[1:2]
    y_rs = stats_ref[2:3]
    y_fill = stats_ref[3:4]
    col0 = jax.lax.broadcasted_iota(jnp.int32, (TL, D), 1) == 0

    mrows = jax.lax.broadcasted_iota(jnp.int32, (1, 1, TL), 2) + start
    m_ref[...] = (mrows < slen).astype(jnp.float32)

    @pl.when(start + TL <= slen)
    def _():
        xb = x_ref[0]
        yb = y_ref[0]
        xo_ref[0] = xb * x_rs + x_fill
        ym = jnp.where(col0, xb - yb, yb)
        yo_ref[0] = ym * y_rs + y_fill

    @pl.when((start < slen) & (start + TL > slen))
    def _():
        xb = x_ref[0]
        yb = y_ref[0]
        rows = jax.lax.broadcasted_iota(jnp.int32, (TL, 1), 0) + start
        valid = rows < slen
        xo_ref[0] = jnp.where(valid, xb * x_rs + x_fill,
                              jnp.broadcast_to(x_fill, (TL, D)))
        ym = jnp.where(col0, xb - yb, yb)
        yo_ref[0] = jnp.where(valid, ym * y_rs + y_fill,
                              jnp.broadcast_to(y_fill, (TL, D)))

    @pl.when(start >= slen)
    def _():
        xo_ref[0] = jnp.broadcast_to(x_fill, (TL, D))
        yo_ref[0] = jnp.broadcast_to(y_fill, (TL, D))


def kernel(x, y, seq_len):
    seq32 = seq_len.astype(jnp.int32)

    stats = pl.pallas_call(
        _stats_kernel,
        grid_spec=pltpu.PrefetchScalarGridSpec(
            num_scalar_prefetch=1,
            grid=(B, NB),
            in_specs=[
                pl.BlockSpec((1, TL, D), _in_index_map),
                pl.BlockSpec((1, TL, D), _in_index_map),
            ],
            out_specs=pl.BlockSpec((8, D), lambda b, j, seq_ref: (0, 0)),
            scratch_shapes=[pltpu.VMEM((4, 1, D), jnp.float32)],
        ),
        out_shape=jax.ShapeDtypeStruct((8, D), jnp.float32),
        compiler_params=pltpu.CompilerParams(
            dimension_semantics=("arbitrary", "arbitrary")),
    )(seq32, x, y)

    x_out, y_out, mask_f = pl.pallas_call(
        _norm_kernel,
        grid_spec=pltpu.PrefetchScalarGridSpec(
            num_scalar_prefetch=1,
            grid=(B, NB),
            in_specs=[
                pl.BlockSpec((8, D), lambda b, j, seq_ref: (0, 0)),
                pl.BlockSpec((1, TL, D), _in_index_map),
                pl.BlockSpec((1, TL, D), _in_index_map),
            ],
            out_specs=[
                pl.BlockSpec((1, TL, D), lambda b, j, seq_ref: (b, j, 0)),
                pl.BlockSpec((1, TL, D), lambda b, j, seq_ref: (b, j, 0)),
                pl.BlockSpec((1, 1, TL),
                             lambda b, j, seq_ref: (b * NB + j, 0, 0)),
            ],
        ),
        out_shape=[
            jax.ShapeDtypeStruct((B, L, D), jnp.float32),
            jax.ShapeDtypeStruct((B, L, D), jnp.float32),
            jax.ShapeDtypeStruct((B * NB, 1, TL), jnp.float32),
        ],
        compiler_params=pltpu.CompilerParams(
            dimension_semantics=("parallel", "parallel")),
    )(seq32, stats, x, y)

    mask = mask_f.reshape(B, L).astype(bool)
    return (x_out, y_out, seq_len, mask)


# speedup vs baseline: 2.4872x; 1.2532x over previous
import jax
import jax.numpy as jnp
from jax.experimental import pallas as pl
from jax.experimental.pallas import tpu as pltpu

B, L, D = 16, 4096, 256
TL = 1024
NB = L // TL
N = B * L


def _in_index_map(b, j, seq_ref):
    last_valid = jnp.maximum((seq_ref[b] + TL - 1) // TL - 1, 0)
    return (b, jnp.minimum(j, last_valid), 0)


def _stats_kernel(seq_ref, x_ref, y_ref, stats_ref, acc_ref):
    b = pl.program_id(0)
    j = pl.program_id(1)

    @pl.when((b == 0) & (j == 0))
    def _():
        acc_ref[...] = jnp.zeros_like(acc_ref)

    start = j * TL
    slen = seq_ref[b]
    col0 = jax.lax.broadcasted_iota(jnp.int32, (TL, D), 1) == 0

    def accumulate(xm, ym):
        acc_ref[0] += jnp.sum(xm, axis=0, keepdims=True)
        acc_ref[1] += jnp.sum(xm * xm, axis=0, keepdims=True)
        acc_ref[2] += jnp.sum(ym, axis=0, keepdims=True)
        acc_ref[3] += jnp.sum(ym * ym, axis=0, keepdims=True)

    @pl.when(start + TL <= slen)
    def _():
        xb = x_ref[0]
        yb = y_ref[0]
        ym = jnp.where(col0, xb - yb, yb)
        accumulate(xb, ym)

    @pl.when((start < slen) & (start + TL > slen))
    def _():
        xb = x_ref[0]
        yb = y_ref[0]
        rows = jax.lax.broadcasted_iota(jnp.int32, (TL, 1), 0) + start
        valid = rows < slen
        xm = jnp.where(valid, xb, 0.0)
        ym = jnp.where(valid, yb, 0.0)
        ym = jnp.where(col0, xm - ym, ym)
        accumulate(xm, ym)

    @pl.when((b == B - 1) & (j == NB - 1))
    def _():
        inv_n = 1.0 / N
        inv_nm1 = 1.0 / (N - 1)
        x_loc = acc_ref[0] * inv_n
        y_loc = acc_ref[2] * inv_n
        x_var = (acc_ref[1] - N * x_loc * x_loc) * inv_nm1
        y_var = (acc_ref[3] - N * y_loc * y_loc) * inv_nm1
        x_rs = jax.lax.rsqrt(x_var)
        y_rs = jax.lax.rsqrt(y_var)
        stats_ref[...] = jnp.concatenate(
            [x_rs, -x_loc * x_rs, y_rs, -y_loc * y_rs,
             jnp.zeros((4, D), jnp.float32)], axis=0)


def _norm_kernel(seq_ref, stats_ref, x_ref, y_ref, xo_ref, yo_ref, m_ref):
    b = pl.program_id(0)
    j = pl.program_id(1)
    start = j * TL
    slen = seq_ref[b]

    x_rs = stats_ref[0:1]
    x_fill = stats_ref[1:2]
    y_rs = stats_ref[2:3]
    y_fill = stats_ref[3:4]
    col0 = jax.lax.broadcasted_iota(jnp.int32, (TL, D), 1) == 0

    mrows = jax.lax.broadcasted_iota(jnp.int32, (1, 1, TL), 2) + start
    m_ref[...] = (mrows < slen).astype(jnp.float32)

    @pl.when(start + TL <= slen)
    def _():
        xb = x_ref[0]
        yb = y_ref[0]
        xo_ref[0] = xb * x_rs + x_fill
        ym = jnp.where(col0, xb - yb, yb)
        yo_ref[0] = ym * y_rs + y_fill

    @pl.when((start < slen) & (start + TL > slen))
    def _():
        xb = x_ref[0]
        yb = y_ref[0]
        rows = jax.lax.broadcasted_iota(jnp.int32, (TL, 1), 0) + start
        valid = rows < slen
        xo_ref[0] = jnp.where(valid, xb * x_rs + x_fill,
                              jnp.broadcast_to(x_fill, (TL, D)))
        ym = jnp.where(col0, xb - yb, yb)
        yo_ref[0] = jnp.where(valid, ym * y_rs + y_fill,
                              jnp.broadcast_to(y_fill, (TL, D)))

    @pl.when(start >= slen)
    def _():
        xo_ref[0] = jnp.broadcast_to(x_fill, (TL, D))
        yo_ref[0] = jnp.broadcast_to(y_fill, (TL, D))


def kernel(x, y, seq_len):
    seq32 = seq_len.astype(jnp.int32)

    stats = pl.pallas_call(
        _stats_kernel,
        grid_spec=pltpu.PrefetchScalarGridSpec(
            num_scalar_prefetch=1,
            grid=(B, NB),
            in_specs=[
                pl.BlockSpec((1, TL, D), _in_index_map),
                pl.BlockSpec((1, TL, D), _in_index_map),
            ],
            out_specs=pl.BlockSpec((8, D), lambda b, j, seq_ref: (0, 0)),
            scratch_shapes=[pltpu.VMEM((4, 1, D), jnp.float32)],
        ),
        out_shape=jax.ShapeDtypeStruct((8, D), jnp.float32),
        compiler_params=pltpu.CompilerParams(
            dimension_semantics=("arbitrary", "arbitrary")),
    )(seq32, x, y)

    x_out, y_out, mask_f = pl.pallas_call(
        _norm_kernel,
        grid_spec=pltpu.PrefetchScalarGridSpec(
            num_scalar_prefetch=1,
            grid=(B, NB),
            in_specs=[
                pl.BlockSpec((8, D), lambda b, j, seq_ref: (0, 0)),
                pl.BlockSpec((1, TL, D), _in_index_map),
                pl.BlockSpec((1, TL, D), _in_index_map),
            ],
            out_specs=[
                pl.BlockSpec((1, TL, D), lambda b, j, seq_ref: (b, j, 0)),
                pl.BlockSpec((1, TL, D), lambda b, j, seq_ref: (b, j, 0)),
                pl.BlockSpec((1, 1, TL),
                             lambda b, j, seq_ref: (b * NB + j, 0, 0)),
            ],
        ),
        out_shape=[
            jax.ShapeDtypeStruct((B, L, D), jnp.float32),
            jax.ShapeDtypeStruct((B, L, D), jnp.float32),
            jax.ShapeDtypeStruct((B * NB, 1, TL), jnp.float32),
        ],
        compiler_params=pltpu.CompilerParams(
            dimension_semantics=("parallel", "parallel")),
    )(seq32, stats, x, y)

    mask = mask_f.reshape(B, L).astype(bool)
    return (x_out, y_out, seq_len, mask)


# speedup vs baseline: 2.6545x; 1.0673x over previous
import jax
import jax.numpy as jnp
from jax.experimental import pallas as pl
from jax.experimental.pallas import tpu as pltpu

B, L, D = 16, 4096, 256
TL = 2048
NB = L // TL
N = B * L


def _in_index_map(b, j, seq_ref):
    last_valid = jnp.maximum((seq_ref[b] + TL - 1) // TL - 1, 0)
    return (b, jnp.minimum(j, last_valid), 0)


def _stats_kernel(seq_ref, x_ref, y_ref, stats_ref, acc_ref):
    b = pl.program_id(0)
    j = pl.program_id(1)

    @pl.when((b == 0) & (j == 0))
    def _():
        acc_ref[...] = jnp.zeros_like(acc_ref)

    start = j * TL
    slen = seq_ref[b]
    col0 = jax.lax.broadcasted_iota(jnp.int32, (TL, D), 1) == 0

    def accumulate(xm, ym):
        acc_ref[0] += jnp.sum(xm, axis=0, keepdims=True)
        acc_ref[1] += jnp.sum(xm * xm, axis=0, keepdims=True)
        acc_ref[2] += jnp.sum(ym, axis=0, keepdims=True)
        acc_ref[3] += jnp.sum(ym * ym, axis=0, keepdims=True)

    @pl.when(start + TL <= slen)
    def _():
        xb = x_ref[0]
        yb = y_ref[0]
        ym = jnp.where(col0, xb - yb, yb)
        accumulate(xb, ym)

    @pl.when((start < slen) & (start + TL > slen))
    def _():
        xb = x_ref[0]
        yb = y_ref[0]
        rows = jax.lax.broadcasted_iota(jnp.int32, (TL, 1), 0) + start
        valid = rows < slen
        xm = jnp.where(valid, xb, 0.0)
        ym = jnp.where(valid, yb, 0.0)
        ym = jnp.where(col0, xm - ym, ym)
        accumulate(xm, ym)

    @pl.when((b == B - 1) & (j == NB - 1))
    def _():
        inv_n = 1.0 / N
        inv_nm1 = 1.0 / (N - 1)
        x_loc = acc_ref[0] * inv_n
        y_loc = acc_ref[2] * inv_n
        x_var = (acc_ref[1] - N * x_loc * x_loc) * inv_nm1
        y_var = (acc_ref[3] - N * y_loc * y_loc) * inv_nm1
        x_rs = jax.lax.rsqrt(x_var)
        y_rs = jax.lax.rsqrt(y_var)
        stats_ref[...] = jnp.concatenate(
            [x_rs, -x_loc * x_rs, y_rs, -y_loc * y_rs,
             jnp.zeros((4, D), jnp.float32)], axis=0)


def _norm_kernel(seq_ref, stats_ref, x_ref, y_ref, xo_ref, yo_ref, m_ref):
    b = pl.program_id(0)
    j = pl.program_id(1)
    start = j * TL
    slen = seq_ref[b]

    x_rs = stats_ref[0:1]
    x_fill = stats_ref[1:2]
    y_rs = stats_ref[2:3]
    y_fill = stats_ref[3:4]
    col0 = jax.lax.broadcasted_iota(jnp.int32, (TL, D), 1) == 0

    mrows = jax.lax.broadcasted_iota(jnp.int32, (1, 1, TL), 2) + start
    m_ref[...] = (mrows < slen).astype(jnp.float32)

    @pl.when(start + TL <= slen)
    def _():
        xb = x_ref[0]
        yb = y_ref[0]
        xo_ref[0] = xb * x_rs + x_fill
        ym = jnp.where(col0, xb - yb, yb)
        yo_ref[0] = ym * y_rs + y_fill

    @pl.when((start < slen) & (start + TL > slen))
    def _():
        xb = x_ref[0]
        yb = y_ref[0]
        rows = jax.lax.broadcasted_iota(jnp.int32, (TL, 1), 0) + start
        valid = rows < slen
        xo_ref[0] = jnp.where(valid, xb * x_rs + x_fill,
                              jnp.broadcast_to(x_fill, (TL, D)))
        ym = jnp.where(col0, xb - yb, yb)
        yo_ref[0] = jnp.where(valid, ym * y_rs + y_fill,
                              jnp.broadcast_to(y_fill, (TL, D)))

    @pl.when(start >= slen)
    def _():
        xo_ref[0] = jnp.broadcast_to(x_fill, (TL, D))
        yo_ref[0] = jnp.broadcast_to(y_fill, (TL, D))


def kernel(x, y, seq_len):
    seq32 = seq_len.astype(jnp.int32)

    stats = pl.pallas_call(
        _stats_kernel,
        grid_spec=pltpu.PrefetchScalarGridSpec(
            num_scalar_prefetch=1,
            grid=(B, NB),
            in_specs=[
                pl.BlockSpec((1, TL, D), _in_index_map),
                pl.BlockSpec((1, TL, D), _in_index_map),
            ],
            out_specs=pl.BlockSpec((8, D), lambda b, j, seq_ref: (0, 0)),
            scratch_shapes=[pltpu.VMEM((4, 1, D), jnp.float32)],
        ),
        out_shape=jax.ShapeDtypeStruct((8, D), jnp.float32),
        compiler_params=pltpu.CompilerParams(
            dimension_semantics=("arbitrary", "arbitrary")),
    )(seq32, x, y)

    x_out, y_out, mask_f = pl.pallas_call(
        _norm_kernel,
        grid_spec=pltpu.PrefetchScalarGridSpec(
            num_scalar_prefetch=1,
            grid=(B, NB),
            in_specs=[
                pl.BlockSpec((8, D), lambda b, j, seq_ref: (0, 0)),
                pl.BlockSpec((1, TL, D), _in_index_map),
                pl.BlockSpec((1, TL, D), _in_index_map),
            ],
            out_specs=[
                pl.BlockSpec((1, TL, D), lambda b, j, seq_ref: (b, j, 0)),
                pl.BlockSpec((1, TL, D), lambda b, j, seq_ref: (b, j, 0)),
                pl.BlockSpec((1, 1, TL),
                             lambda b, j, seq_ref: (b * NB + j, 0, 0)),
            ],
        ),
        out_shape=[
            jax.ShapeDtypeStruct((B, L, D), jnp.float32),
            jax.ShapeDtypeStruct((B, L, D), jnp.float32),
            jax.ShapeDtypeStruct((B * NB, 1, TL), jnp.float32),
        ],
        compiler_params=pltpu.CompilerParams(
            dimension_semantics=("parallel", "parallel")),
    )(seq32, stats, x, y)

    mask = mask_f.reshape(B, L).astype(bool)
    return (x_out, y_out, seq_len, mask)
